# baseline (speedup 1.0000x reference)
import functools

import jax
import jax.numpy as jnp
from jax import lax
from jax.experimental import pallas as pl
from jax.experimental.pallas import tpu as pltpu

_EPS = 1e-5


def _shift_lanes(x, s):
    if s == 0:
        return x
    rows = x.shape[0]
    z = jnp.zeros((rows, abs(s)), x.dtype)
    if s > 0:
        return jnp.concatenate([x[:, s:], z], axis=1)
    return jnp.concatenate([z, x[:, :s]], axis=1)


def _conv3x3(xin, wall, keep_l, keep_r, width, co):
    zero = jnp.bfloat16(0)
    xm = jnp.where(keep_l, _shift_lanes(xin, -1), zero)
    xp = jnp.where(keep_r, _shift_lanes(xin, 1), zero)
    x3 = jnp.concatenate([xm, xin, xp], axis=0)
    y3 = jnp.dot(wall, x3, preferred_element_type=jnp.float32)
    t0 = y3[0:co]
    t1 = y3[co:2 * co]
    t2 = y3[2 * co:3 * co]
    return t1 + _shift_lanes(t0, -width) + _shift_lanes(t2, width)


def _edge_masks(ci, length, width):
    wpos = lax.broadcasted_iota(jnp.int32, (ci, length), 1) % width
    return wpos != 0, wpos != (width - 1)


def _conv1_stats_kernel(nb, width, x_ref, w1_ref, b1_ref,
                        y1_ref, ssum_ref, ssq_ref):
    ci, length = x_ref.shape[1], x_ref.shape[2]
    co = b1_ref.shape[0]
    keep_l, keep_r = _edge_masks(ci, length, width)
    acc_s = jnp.zeros((co, 1), jnp.float32)
    acc_q = jnp.zeros((co, 1), jnp.float32)
    for i in range(nb):
        xb = x_ref[i].astype(jnp.bfloat16)
        y = _conv3x3(xb, w1_ref[...], keep_l, keep_r, width, co)
        y = y + b1_ref[...]
        y1_ref[i] = y.astype(jnp.bfloat16)
        acc_s = acc_s + jnp.sum(y, axis=1, keepdims=True)
        acc_q = acc_q + jnp.sum(y * y, axis=1, keepdims=True)
    ssum_ref[0] = acc_s
    ssq_ref[0] = acc_q


def _bn_relu_conv2_kernel(nb, width, y1_ref, sc_ref, sh_ref, w2_ref, b2_ref,
                          o_ref):
    co, length = y1_ref.shape[1], y1_ref.shape[2]
    keep_l, keep_r = _edge_masks(co, length, width)
    for i in range(nb):
        a = jnp.maximum(y1_ref[i] * sc_ref[...] + sh_ref[...], 0.0)
        ab = a.astype(jnp.bfloat16)
        o_ref[i] = _conv3x3(ab, w2_ref[...], keep_l, keep_r, width, co) \
            + b2_ref[...]


@jax.jit
def _forward(x_nchw, w1, b1, gamma, beta, w2, b2):
    n, ci, h, w = x_nchw.shape
    co = w1.shape[-1]
    length = h * w

    nb = 16
    while n % nb:
        nb //= 2
    steps = n // nb

    x_r = x_nchw.reshape(n, ci, length).astype(jnp.float32)
    w1a = jnp.transpose(w1.astype(jnp.bfloat16), (0, 3, 1, 2)).reshape(
        3 * co, 3 * ci)
    w2a = jnp.transpose(w2.astype(jnp.bfloat16), (0, 3, 1, 2)).reshape(
        3 * co, 3 * co)
    b1c = b1.astype(jnp.float32).reshape(co, 1)
    b2c = b2.astype(jnp.float32).reshape(co, 1)

    k1 = functools.partial(_conv1_stats_kernel, nb, w)
    y1, ssum, ssq = pl.pallas_call(
        k1,
        out_shape=(jax.ShapeDtypeStruct((n, co, length), jnp.bfloat16),
                   jax.ShapeDtypeStruct((steps, co, 1), jnp.float32),
                   jax.ShapeDtypeStruct((steps, co, 1), jnp.float32)),
        grid=(steps,),
        in_specs=[
            pl.BlockSpec((nb, ci, length), lambda i: (i, 0, 0)),
            pl.BlockSpec((3 * co, 3 * ci), lambda i: (0, 0)),
            pl.BlockSpec((co, 1), lambda i: (0, 0)),
        ],
        out_specs=(
            pl.BlockSpec((nb, co, length), lambda i: (i, 0, 0)),
            pl.BlockSpec((1, co, 1), lambda i: (i, 0, 0)),
            pl.BlockSpec((1, co, 1), lambda i: (i, 0, 0)),
        ),
        compiler_params=pltpu.CompilerParams(
            dimension_semantics=("parallel",)),
    )(x_r, w1a, b1c)

    cnt = float(n * h * w)
    s_c = jnp.sum(ssum[:, :, 0], axis=0)
    q_c = jnp.sum(ssq[:, :, 0], axis=0)
    mean = s_c / cnt
    var = jnp.maximum(q_c / cnt - mean * mean, 0.0)
    scale = gamma.astype(jnp.float32) * lax.rsqrt(var + _EPS)
    shift = beta.astype(jnp.float32) - mean * scale
    sc_col = scale.reshape(co, 1)
    sh_col = shift.reshape(co, 1)

    k2 = functools.partial(_bn_relu_conv2_kernel, nb, w)
    out = pl.pallas_call(
        k2,
        out_shape=jax.ShapeDtypeStruct((n, co, length), jnp.float32),
        grid=(steps,),
        in_specs=[
            pl.BlockSpec((nb, co, length), lambda i: (i, 0, 0)),
            pl.BlockSpec((co, 1), lambda i: (0, 0)),
            pl.BlockSpec((co, 1), lambda i: (0, 0)),
            pl.BlockSpec((3 * co, 3 * co), lambda i: (0, 0)),
            pl.BlockSpec((co, 1), lambda i: (0, 0)),
        ],
        out_specs=pl.BlockSpec((nb, co, length), lambda i: (i, 0, 0)),
        compiler_params=pltpu.CompilerParams(
            dimension_semantics=("parallel",)),
    )(y1, sc_col, sh_col, w2a, b2c)

    return out.reshape(n, co, h, w)


def kernel(x_nchw, w1, b1, gamma, beta, w2, b2):
    return _forward(x_nchw, w1, b1, gamma, beta, w2, b2)


# speedup vs baseline: 1.2534x; 1.2534x over previous
import functools

import jax
import jax.numpy as jnp
from jax import lax
from jax.experimental import pallas as pl
from jax.experimental.pallas import tpu as pltpu

_EPS = 1e-5


def _shift_lanes(x, s):
    if s == 0:
        return x
    rows = x.shape[0]
    z = jnp.zeros((rows, abs(s)), x.dtype)
    if s > 0:
        return jnp.concatenate([x[:, s:], z], axis=1)
    return jnp.concatenate([z, x[:, :s]], axis=1)


def _conv3x3(xin, wall, keep_l, keep_r, width, co):
    zero = jnp.bfloat16(0)
    xm = jnp.where(keep_l, _shift_lanes(xin, -1), zero)
    xp = jnp.where(keep_r, _shift_lanes(xin, 1), zero)
    x3 = jnp.concatenate([xm, xin, xp], axis=0)
    y3 = jnp.dot(wall, x3, preferred_element_type=jnp.float32)
    t0 = y3[0:co]
    t1 = y3[co:2 * co]
    t2 = y3[2 * co:3 * co]
    return t1 + _shift_lanes(t0, -width) + _shift_lanes(t2, width)


def _edge_masks(ci, length, width):
    wpos = lax.broadcasted_iota(jnp.int32, (ci, length), 1) % width
    return wpos != 0, wpos != (width - 1)


def _conv1_stats_kernel(nb, width, x_ref, w1_ref, b1_ref,
                        y1_ref, ssum_ref, ssq_ref):
    ci, length = x_ref.shape[1], x_ref.shape[2]
    co = b1_ref.shape[0]
    keep_l, keep_r = _edge_masks(ci, length, width)
    acc_s = jnp.zeros((co, 1), jnp.float32)
    acc_q = jnp.zeros((co, 1), jnp.float32)
    for i in range(nb):
        xb = x_ref[i].astype(jnp.bfloat16)
        y = _conv3x3(xb, w1_ref[...], keep_l, keep_r, width, co)
        y = y + b1_ref[...]
        y1_ref[i] = y.astype(jnp.bfloat16)
        acc_s = acc_s + jnp.sum(y, axis=1, keepdims=True)
        acc_q = acc_q + jnp.sum(y * y, axis=1, keepdims=True)
    ssum_ref[0] = acc_s
    ssq_ref[0] = acc_q


def _bn_relu_conv2_kernel(nb, width, y1_ref, sc_ref, sh_ref, w2_ref, b2_ref,
                          o_ref):
    co, length = y1_ref.shape[1], y1_ref.shape[2]
    keep_l, keep_r = _edge_masks(co, length, width)
    for i in range(nb):
        a = jnp.maximum(y1_ref[i] * sc_ref[...] + sh_ref[...], 0.0)
        ab = a.astype(jnp.bfloat16)
        o_ref[i] = _conv3x3(ab, w2_ref[...], keep_l, keep_r, width, co) \
            + b2_ref[...]


@jax.jit
def _forward(x_nchw, w1, b1, gamma, beta, w2, b2):
    n, ci, h, w = x_nchw.shape
    co = w1.shape[-1]
    length = h * w

    nb = 4
    while n % nb:
        nb //= 2
    steps = n // nb

    x_r = x_nchw.reshape(n, ci, length).astype(jnp.float32)
    w1a = jnp.transpose(w1.astype(jnp.bfloat16), (0, 3, 1, 2)).reshape(
        3 * co, 3 * ci)
    w2a = jnp.transpose(w2.astype(jnp.bfloat16), (0, 3, 1, 2)).reshape(
        3 * co, 3 * co)
    b1c = b1.astype(jnp.float32).reshape(co, 1)
    b2c = b2.astype(jnp.float32).reshape(co, 1)

    k1 = functools.partial(_conv1_stats_kernel, nb, w)
    y1, ssum, ssq = pl.pallas_call(
        k1,
        out_shape=(jax.ShapeDtypeStruct((n, co, length), jnp.bfloat16),
                   jax.ShapeDtypeStruct((steps, co, 1), jnp.float32),
                   jax.ShapeDtypeStruct((steps, co, 1), jnp.float32)),
        grid=(steps,),
        in_specs=[
            pl.BlockSpec((nb, ci, length), lambda i: (i, 0, 0)),
            pl.BlockSpec((3 * co, 3 * ci), lambda i: (0, 0)),
            pl.BlockSpec((co, 1), lambda i: (0, 0)),
        ],
        out_specs=(
            pl.BlockSpec((nb, co, length), lambda i: (i, 0, 0)),
            pl.BlockSpec((1, co, 1), lambda i: (i, 0, 0)),
            pl.BlockSpec((1, co, 1), lambda i: (i, 0, 0)),
        ),
        compiler_params=pltpu.CompilerParams(
            dimension_semantics=("parallel",)),
    )(x_r, w1a, b1c)

    cnt = float(n * h * w)
    s_c = jnp.sum(ssum[:, :, 0], axis=0)
    q_c = jnp.sum(ssq[:, :, 0], axis=0)
    mean = s_c / cnt
    var = jnp.maximum(q_c / cnt - mean * mean, 0.0)
    scale = gamma.astype(jnp.float32) * lax.rsqrt(var + _EPS)
    shift = beta.astype(jnp.float32) - mean * scale
    sc_col = scale.reshape(co, 1)
    sh_col = shift.reshape(co, 1)

    k2 = functools.partial(_bn_relu_conv2_kernel, nb, w)
    out = pl.pallas_call(
        k2,
        out_shape=jax.ShapeDtypeStruct((n, co, length), jnp.float32),
        grid=(steps,),
        in_specs=[
            pl.BlockSpec((nb, co, length), lambda i: (i, 0, 0)),
            pl.BlockSpec((co, 1), lambda i: (0, 0)),
            pl.BlockSpec((co, 1), lambda i: (0, 0)),
            pl.BlockSpec((3 * co, 3 * co), lambda i: (0, 0)),
            pl.BlockSpec((co, 1), lambda i: (0, 0)),
        ],
        out_specs=pl.BlockSpec((nb, co, length), lambda i: (i, 0, 0)),
        compiler_params=pltpu.CompilerParams(
            dimension_semantics=("parallel",)),
    )(y1, sc_col, sh_col, w2a, b2c)

    return out.reshape(n, co, h, w)


def kernel(x_nchw, w1, b1, gamma, beta, w2, b2):
    return _forward(x_nchw, w1, b1, gamma, beta, w2, b2)
